# Initial kernel scaffold; baseline (speedup 1.0000x reference)
import jax
import jax.numpy as jnp
from jax.experimental import pallas as pl


def kernel(uEmbeds, entiEmbs, att, item_entities):
    raise NotImplementedError("write your pallas kernel here")


# speedup vs baseline: 9.7126x; 9.7126x over previous
import functools

import jax
import jax.numpy as jnp
from jax import lax
from jax.experimental import pallas as pl
from jax.experimental.pallas import tpu as pltpu
from jax.experimental.pallas import tpu_sc as plsc

_NW = 32
_C = 64
_SEG = 128
_L = 16


def _build_kernel(n_pad, epi, d, table_rows):
    K = n_pad // (_NW * _C)
    nseg = (_C * epi) // _SEG

    mesh = plsc.VectorSubcoreMesh(core_axis_name="c", subcore_axis_name="s")

    @functools.partial(
        pl.kernel,
        out_type=jax.ShapeDtypeStruct((n_pad, d), jnp.float32),
        mesh=mesh,
        compiler_params=pltpu.CompilerParams(use_tc_tiling_on_sc=False),
        scratch_types=[
            pltpu.VMEM((_C * epi,), jnp.int32),
            pltpu.VMEM((_C * epi,), jnp.float32),
            pltpu.VMEM((_C * epi, d), jnp.float32),
            pltpu.VMEM((_C, d), jnp.float32),
            pltpu.VMEM((_C, d), jnp.float32),
            pltpu.SemaphoreType.DMA,
        ],
    )
    def pooled(table, idxf, attf, out_hbm,
               idx_v, att_v, rows_v, base_v, out_v, sem):
        wid = lax.axis_index("s") * 2 + lax.axis_index("c")

        def chunk_body(k, carry):
            g = wid * K + k
            base = g * _C

            pltpu.sync_copy(idxf.at[pl.ds(base * epi, _C * epi)], idx_v)
            pltpu.sync_copy(attf.at[pl.ds(base * epi, _C * epi)], att_v)
            pltpu.sync_copy(table.at[pl.ds(base, _C)], base_v)

            descs = []
            for j in range(nseg):
                descs.append(pltpu.async_copy(
                    table.at[idx_v.at[pl.ds(j * _SEG, _SEG)]],
                    rows_v.at[pl.ds(j * _SEG, _SEG)], sem))
            for desc in descs:
                desc.wait()

            def block_body(ib, carry):
                i0 = ib * _L
                logits = [att_v[pl.ds(e * _C + i0, _L)] for e in range(epi)]
                m = functools.reduce(jnp.maximum, logits)
                probs = [jnp.exp(v - m) for v in logits]
                s = functools.reduce(jnp.add, probs)
                inv = 1.0 / s
                w = [p * inv for p in probs]
                for lane in range(_L):
                    i = i0 + lane
                    a0 = base_v[i, pl.ds(0, _L)]
                    a1 = base_v[i, pl.ds(_L, _L)]
                    r = i * epi
                    for e in range(epi):
                        ws = w[e][lane]
                        a0 = a0 + ws * rows_v[r + e, pl.ds(0, _L)]
                        a1 = a1 + ws * rows_v[r + e, pl.ds(_L, _L)]
                    out_v[i, pl.ds(0, _L)] = a0
                    out_v[i, pl.ds(_L, _L)] = a1
                return carry

            lax.fori_loop(0, _C // _L, block_body, 0)

            pltpu.sync_copy(out_v, out_hbm.at[pl.ds(base, _C)])
            return carry

        lax.fori_loop(0, K, chunk_body, 0)

    return pooled


def kernel(uEmbeds, entiEmbs, att, item_entities):
    n, epi = att.shape
    d = entiEmbs.shape[1]
    per_block = _NW * _C
    n_pad = ((n + per_block - 1) // per_block) * per_block
    pad = n_pad - n

    idx_flat = item_entities.astype(jnp.int32).reshape(-1)
    att_pad = att.astype(jnp.float32)
    if pad:
        idx_flat = jnp.pad(idx_flat, (0, pad * epi))
        att_pad = jnp.pad(att_pad, ((0, pad), (0, 0)))
    att_flat = att_pad.reshape(-1, _C, epi).transpose(0, 2, 1).reshape(-1)
    pooled = _build_kernel(n_pad, epi, d, entiEmbs.shape[0])
    out = pooled(entiEmbs, idx_flat, att_flat)
    return (uEmbeds, out[:n])
